# baseline (speedup 1.0000x reference)
import functools

import jax
import jax.numpy as jnp
from jax import lax
from jax.experimental import pallas as pl
from jax.experimental.pallas import tpu as pltpu
from jax.experimental.pallas import tpu_sc as plsc

BETA = 0.25
BB = 256


def _stage1_body(nb, k_total, z_ref, cb_ref, tok_ref, flat_ref, loss_ref,
                 util_ref, sqw_ref, used_ref):
    t = pl.program_id(0)
    b = pl.program_id(1)
    z = z_ref[...]
    cb = cb_ref[0]

    @pl.when((t == 0) & (b == 0))
    def _():
        loss_ref[0, 0] = 0.0
        util_ref[0, 0] = 0.0

    @pl.when(b == 0)
    def _():
        sqw_ref[0, :] = jnp.sum(cb * cb, axis=1)

    cross = lax.dot_general(z, cb, (((1,), (1,)), ((), ())),
                            preferred_element_type=jnp.float32)
    sq_z = jnp.sum(z * z, axis=1, keepdims=True)
    dist = (sq_z - 2.0 * cross) + sqw_ref[0, :][None, :]
    min_d = jnp.min(dist, axis=1, keepdims=True)
    kiota = lax.broadcasted_iota(jnp.int32, dist.shape, 1)
    tok = jnp.min(jnp.where(dist == min_d, kiota, k_total), axis=1)
    tok_ref[0, 0, :] = tok
    flat_ref[0, 0, :] = tok + t * k_total

    loss_ref[0, 0] += jnp.sum(min_d)

    onehot = jnp.where(tok[:, None] == kiota, 1, 0)
    cur = jnp.sum(onehot, axis=0)

    @pl.when(b == 0)
    def _():
        used_ref[0, :] = cur

    @pl.when(b != 0)
    def _():
        used_ref[0, :] = used_ref[0, :] + cur

    @pl.when(b == nb - 1)
    def _():
        util_ref[0, 0] += jnp.sum((used_ref[0, :] > 0).astype(jnp.float32))


def _make_stage1(B, T, K, D):
    nb = B // BB
    body = functools.partial(_stage1_body, nb, K)
    return pl.pallas_call(
        body,
        grid=(T, nb),
        in_specs=[
            pl.BlockSpec((BB, D), lambda t, b: (b, t)),
            pl.BlockSpec((1, K, D), lambda t, b: (t, 0, 0)),
        ],
        out_specs=[
            pl.BlockSpec((1, 1, BB), lambda t, b: (t * nb + b, 0, 0)),
            pl.BlockSpec((1, 1, BB), lambda t, b: (t * nb + b, 0, 0)),
            pl.BlockSpec((1, 1), lambda t, b: (0, 0),
                         memory_space=pltpu.SMEM),
            pl.BlockSpec((1, 1), lambda t, b: (0, 0),
                         memory_space=pltpu.SMEM),
        ],
        out_shape=[
            jax.ShapeDtypeStruct((T * nb, 1, BB), jnp.int32),
            jax.ShapeDtypeStruct((T * nb, 1, BB), jnp.int32),
            jax.ShapeDtypeStruct((1, 1), jnp.float32),
            jax.ShapeDtypeStruct((1, 1), jnp.float32),
        ],
        scratch_shapes=[
            pltpu.VMEM((1, K), jnp.float32),
            pltpu.VMEM((1, K), jnp.int32),
        ],
    )


def _make_sc_gather(n_rows, D, nw, chunks, chunk):
    mesh = plsc.VectorSubcoreMesh(core_axis_name="c", subcore_axis_name="s")
    info = plsc.get_sparse_core_info()
    nc = info.num_cores
    cpw = chunks * chunk

    @functools.partial(
        pl.kernel, mesh=mesh,
        out_type=jax.ShapeDtypeStruct((n_rows, D), jnp.float32),
        scratch_types=[
            pltpu.VMEM((chunks, chunk), jnp.int32),
            pltpu.VMEM((chunk, D), jnp.float32),
            pltpu.SemaphoreType.DMA,
        ],
    )
    def gk(table_hbm, idx_hbm, out_hbm, idx_v, rows_v, sem):
        wid = lax.axis_index("s") * nc + lax.axis_index("c")
        pltpu.sync_copy(idx_hbm.at[wid], idx_v)
        for j in range(chunks):
            pltpu.async_copy(table_hbm.at[idx_v.at[j]], rows_v, sem).wait()
            pltpu.sync_copy(rows_v, out_hbm.at[pl.ds(wid * cpw + j * chunk,
                                                     chunk)])

    return gk


def kernel(z_e, codebooks):
    B, T, D = z_e.shape
    _, K, _ = codebooks.shape
    nb = B // BB

    tok3, flat3, loss, util = _make_stage1(B, T, K, D)(
        z_e.reshape(B, T * D), codebooks)

    tokens = tok3.reshape(T, B).T
    flat_bt = flat3.reshape(T, B).T

    info = plsc.get_sparse_core_info()
    nw = info.num_cores * info.num_subcores
    n_rows = B * T
    chunk = 128
    chunks = n_rows // (nw * chunk)
    idx = flat_bt.reshape(nw, chunks, chunk)

    zq_flat = _make_sc_gather(n_rows, D, nw, chunks, chunk)(
        codebooks.reshape(T * K, D), idx)
    z_q = zq_flat.reshape(B, T, D)

    vq_loss = loss[0, 0] * jnp.float32(BETA / (B * T * D))
    utilization = util[0, 0] / jnp.float32(T * K)
    return z_q, tokens, vq_loss, utilization
